# Initial kernel scaffold; baseline (speedup 1.0000x reference)
import functools
import jax
import jax.numpy as jnp
from jax import lax
from jax.experimental import pallas as pl
from jax.experimental.pallas import tpu as pltpu
from jax.experimental.pallas import tpu_sc as plsc

N = 10000
E = 320000
D = 128
NCLS = 40

NC = 2
NS = 16
NW = NC * NS
CH = 128
STEPS = 80
E_PAD = NW * STEPS * CH
N_PAD = 10112
ROWS_PER_TILE = N_PAD // NS
TCB = 1000


def _pre_body(x_ref, wpT, bp, wlT, wrT, z_ref, r_ref):
    h = jnp.maximum(jnp.dot(x_ref[...], wpT[...],
                            preferred_element_type=jnp.float32) + bp[...], 0.0)
    z_ref[...] = jnp.dot(h, wlT[...], preferred_element_type=jnp.float32)
    r_ref[...] = jnp.dot(h, wrT[...], preferred_element_type=jnp.float32)


def _mid_body(p_ref, c_ref, r_ref, bl, wlT, wrT, z_ref, rn_ref):
    p = p_ref[0] + p_ref[1]
    c = c_ref[0, :, 0:1] + c_ref[1, :, 0:1]
    inv = 1.0 / jnp.maximum(c, 1.0)
    h = jnp.maximum(p * inv + bl[...] + r_ref[...], 0.0)
    z_ref[...] = jnp.dot(h, wlT[...], preferred_element_type=jnp.float32)
    rn_ref[...] = jnp.dot(h, wrT[...], preferred_element_type=jnp.float32)


def _fin_body(p_ref, c_ref, r_ref, bl, wcT, bc, o_ref):
    p = p_ref[0] + p_ref[1]
    c = c_ref[0, :, 0:1] + c_ref[1, :, 0:1]
    inv = 1.0 / jnp.maximum(c, 1.0)
    h = jnp.maximum(p * inv + bl[...] + r_ref[...], 0.0)
    o_ref[...] = jnp.dot(h, wcT[...], preferred_element_type=jnp.float32) + bc[...]


def _row_spec():
    return pl.BlockSpec((TCB, D), lambda i: (i, 0))


def _full_spec(shape):
    return pl.BlockSpec(shape, lambda i: tuple(0 for _ in shape))


def _part_spec():
    return pl.BlockSpec((2, TCB, D), lambda i: (0, i, 0))


def _cnt_spec():
    return pl.BlockSpec((2, TCB, 16), lambda i: (0, i, 0))


_GRID = N // TCB


def _tc_pre(x, wpT, bp, wlT, wrT):
    return pl.pallas_call(
        _pre_body,
        grid=(_GRID,),
        in_specs=[_row_spec(), _full_spec((D, D)), _full_spec((1, D)),
                  _full_spec((D, D)), _full_spec((D, D))],
        out_specs=[_row_spec(), _row_spec()],
        out_shape=[jax.ShapeDtypeStruct((N, D), jnp.float32),
                   jax.ShapeDtypeStruct((N, D), jnp.float32)],
    )(x, wpT, bp, wlT, wrT)


def _tc_mid(parts, cnts, r, bl, wlT, wrT):
    return pl.pallas_call(
        _mid_body,
        grid=(_GRID,),
        in_specs=[_part_spec(), _cnt_spec(), _row_spec(), _full_spec((1, D)),
                  _full_spec((D, D)), _full_spec((D, D))],
        out_specs=[_row_spec(), _row_spec()],
        out_shape=[jax.ShapeDtypeStruct((N, D), jnp.float32),
                   jax.ShapeDtypeStruct((N, D), jnp.float32)],
    )(parts, cnts, r, bl, wlT, wrT)


def _tc_fin(parts, cnts, r, bl, wcT, bc):
    return pl.pallas_call(
        _fin_body,
        grid=(_GRID,),
        in_specs=[_part_spec(), _cnt_spec(), _row_spec(), _full_spec((1, D)),
                  _full_spec((D, D)), _full_spec((1, D))],
        out_specs=_row_spec(),
        out_shape=jax.ShapeDtypeStruct((N, D), jnp.float32),
    )(parts, cnts, r, bl, wcT, bc)


def _sc_agg_body(with_cnt, *refs):
    if with_cnt:
        (z, srcr, dstr, zbig, zcnt, ones_h,
         p_out, c_out,
         acc, cnt_sh, sbuf, dbuf, rows, ones_v, sem) = refs
    else:
        (z, srcr, dstr, zbig,
         p_out,
         acc, sbuf, dbuf, rows, sem) = refs

    cid = lax.axis_index("c")
    sid = lax.axis_index("s")
    wid = cid * NS + sid
    base = sid * ROWS_PER_TILE

    pltpu.sync_copy(srcr.at[wid], sbuf)
    pltpu.sync_copy(dstr.at[wid], dbuf)
    pltpu.sync_copy(zbig.at[pl.ds(base, ROWS_PER_TILE)],
                    acc.at[pl.ds(base, ROWS_PER_TILE)])
    if with_cnt:
        pltpu.sync_copy(zcnt.at[pl.ds(base, ROWS_PER_TILE)],
                        cnt_sh.at[pl.ds(base, ROWS_PER_TILE)])
        pltpu.sync_copy(ones_h, ones_v)
    plsc.subcore_barrier()

    def step(j, carry):
        pltpu.async_copy(z.at[sbuf.at[j]], rows, sem).wait()
        pltpu.sync_copy(rows, acc.at[dbuf.at[j]], add=True)
        if with_cnt:
            pltpu.sync_copy(ones_v, cnt_sh.at[dbuf.at[j]], add=True)
        return carry

    lax.fori_loop(0, STEPS, step, 0)
    plsc.subcore_barrier()

    pltpu.sync_copy(acc.at[pl.ds(base, ROWS_PER_TILE)],
                    p_out.at[cid, pl.ds(base, ROWS_PER_TILE)])
    if with_cnt:
        pltpu.sync_copy(cnt_sh.at[pl.ds(base, ROWS_PER_TILE)],
                        c_out.at[cid, pl.ds(base, ROWS_PER_TILE)])


def _make_sc_agg(with_cnt):
    mesh = plsc.VectorSubcoreMesh(core_axis_name="c", subcore_axis_name="s")
    out_type = [jax.ShapeDtypeStruct((NC, N_PAD, D), jnp.float32)]
    scratch = [
        pltpu.VMEM_SHARED((N_PAD, D), jnp.float32),
    ]
    if with_cnt:
        out_type.append(jax.ShapeDtypeStruct((NC, N_PAD, 16), jnp.float32))
        scratch.append(pltpu.VMEM_SHARED((N_PAD, 16), jnp.float32))
    scratch += [
        pltpu.VMEM((STEPS, CH), jnp.int32),
        pltpu.VMEM((STEPS, CH), jnp.int32),
        pltpu.VMEM((CH, D), jnp.float32),
    ]
    if with_cnt:
        scratch.append(pltpu.VMEM((CH, 16), jnp.float32))
    scratch.append(pltpu.SemaphoreType.DMA)

    return pl.kernel(
        functools.partial(_sc_agg_body, with_cnt),
        out_type=out_type,
        mesh=mesh,
        scratch_types=scratch,
    )


_sc_agg_cnt = _make_sc_agg(True)
_sc_agg = _make_sc_agg(False)


def kernel(x, edge_index, W_proj, b_proj, Wl1, bl1, Wr1, Wl2, bl2, Wr2,
           Wl3, bl3, Wr3, W_cls, b_cls):
    f32 = jnp.float32
    src = edge_index[0]
    dst = edge_index[1]
    pad = E_PAD - E
    srcr = jnp.concatenate([src, jnp.zeros((pad,), jnp.int32)]
                           ).reshape(NW, STEPS, CH)
    dstr = jnp.concatenate([dst, jnp.full((pad,), N, jnp.int32)]
                           ).reshape(NW, STEPS, CH)

    zbig = jnp.zeros((N_PAD, D), f32)
    zcnt = jnp.zeros((N_PAD, 16), f32)
    ones_h = jnp.ones((CH, 16), f32)

    wpT = W_proj.T
    bp = b_proj.reshape(1, D)
    wl1T, wr1T = Wl1.T, Wr1.T
    wl2T, wr2T = Wl2.T, Wr2.T
    wl3T, wr3T = Wl3.T, Wr3.T
    bl1r, bl2r, bl3r = bl1.reshape(1, D), bl2.reshape(1, D), bl3.reshape(1, D)
    wcT = jnp.zeros((D, D), f32).at[:, :NCLS].set(W_cls.T)
    bc = jnp.zeros((1, D), f32).at[0, :NCLS].set(b_cls)

    z1, r1 = _tc_pre(x, wpT, bp, wl1T, wr1T)
    p1, cnts = _sc_agg_cnt(z1, srcr, dstr, zbig, zcnt, ones_h)
    z2, r2 = _tc_mid(p1, cnts, r1, bl1r, wl2T, wr2T)
    (p2,) = _sc_agg(z2, srcr, dstr, zbig)
    z3, r3 = _tc_mid(p2, cnts, r2, bl2r, wl3T, wr3T)
    (p3,) = _sc_agg(z3, srcr, dstr, zbig)
    out_pad = _tc_fin(p3, cnts, r3, bl3r, wcT, bc)
    return out_pad[:, :NCLS]


# speedup vs baseline: 3.0359x; 3.0359x over previous
import functools
import jax
import jax.numpy as jnp
from jax import lax
from jax.experimental import pallas as pl
from jax.experimental.pallas import tpu as pltpu
from jax.experimental.pallas import tpu_sc as plsc

N = 10000
E = 320000
D = 128
NCLS = 40

NC = 2
NS = 16
NW = NC * NS
CH = 128
STEPS = 80
IB = 16
NSUP = STEPS // IB
E_PAD = NW * STEPS * CH
N_PAD = 10112
ROWS_PER_TILE = N_PAD // NS
TCB = 1000


def _pre_body(x_ref, wpT, bp, wlT, wrT, z_ref, r_ref):
    h = jnp.maximum(jnp.dot(x_ref[...], wpT[...],
                            preferred_element_type=jnp.float32) + bp[...], 0.0)
    z_ref[...] = jnp.dot(h, wlT[...], preferred_element_type=jnp.float32)
    r_ref[...] = jnp.dot(h, wrT[...], preferred_element_type=jnp.float32)


def _mid_body(p_ref, c_ref, r_ref, bl, wlT, wrT, z_ref, rn_ref):
    p = p_ref[0] + p_ref[1]
    c = c_ref[0, :, 0:1] + c_ref[1, :, 0:1]
    inv = 1.0 / jnp.maximum(c, 1.0)
    h = jnp.maximum(p * inv + bl[...] + r_ref[...], 0.0)
    z_ref[...] = jnp.dot(h, wlT[...], preferred_element_type=jnp.float32)
    rn_ref[...] = jnp.dot(h, wrT[...], preferred_element_type=jnp.float32)


def _fin_body(p_ref, c_ref, r_ref, bl, wcT, bc, o_ref):
    p = p_ref[0] + p_ref[1]
    c = c_ref[0, :, 0:1] + c_ref[1, :, 0:1]
    inv = 1.0 / jnp.maximum(c, 1.0)
    h = jnp.maximum(p * inv + bl[...] + r_ref[...], 0.0)
    o_ref[...] = jnp.dot(h, wcT[...], preferred_element_type=jnp.float32) + bc[...]


def _row_spec():
    return pl.BlockSpec((TCB, D), lambda i: (i, 0))


def _full_spec(shape):
    return pl.BlockSpec(shape, lambda i: tuple(0 for _ in shape))


def _part_spec():
    return pl.BlockSpec((2, TCB, D), lambda i: (0, i, 0))


def _cnt_spec():
    return pl.BlockSpec((2, TCB, D), lambda i: (0, i, 0))


_GRID = N // TCB


def _tc_pre(x, wpT, bp, wlT, wrT):
    return pl.pallas_call(
        _pre_body,
        grid=(_GRID,),
        in_specs=[_row_spec(), _full_spec((D, D)), _full_spec((1, D)),
                  _full_spec((D, D)), _full_spec((D, D))],
        out_specs=[_row_spec(), _row_spec()],
        out_shape=[jax.ShapeDtypeStruct((N, D), jnp.float32),
                   jax.ShapeDtypeStruct((N, D), jnp.float32)],
    )(x, wpT, bp, wlT, wrT)


def _tc_mid(parts, cnts, r, bl, wlT, wrT):
    return pl.pallas_call(
        _mid_body,
        grid=(_GRID,),
        in_specs=[_part_spec(), _cnt_spec(), _row_spec(), _full_spec((1, D)),
                  _full_spec((D, D)), _full_spec((D, D))],
        out_specs=[_row_spec(), _row_spec()],
        out_shape=[jax.ShapeDtypeStruct((N, D), jnp.float32),
                   jax.ShapeDtypeStruct((N, D), jnp.float32)],
    )(parts, cnts, r, bl, wlT, wrT)


def _tc_fin(parts, cnts, r, bl, wcT, bc):
    return pl.pallas_call(
        _fin_body,
        grid=(_GRID,),
        in_specs=[_part_spec(), _cnt_spec(), _row_spec(), _full_spec((1, D)),
                  _full_spec((D, D)), _full_spec((1, D))],
        out_specs=_row_spec(),
        out_shape=jax.ShapeDtypeStruct((N, D), jnp.float32),
    )(parts, cnts, r, bl, wcT, bc)


def _sc_agg_body(z, srcr, dstr, zbig, p_out, acc, sbuf, dbuf, rows, sem):
    cid = lax.axis_index("c")
    sid = lax.axis_index("s")
    wid = cid * NS + sid
    base = sid * ROWS_PER_TILE

    pltpu.sync_copy(zbig.at[pl.ds(base, ROWS_PER_TILE)],
                    acc.at[pl.ds(base, ROWS_PER_TILE)])
    plsc.subcore_barrier()

    def super_step(g, carry):
        pltpu.sync_copy(srcr.at[wid, pl.ds(g * IB, IB)], sbuf)
        pltpu.sync_copy(dstr.at[wid, pl.ds(g * IB, IB)], dbuf)

        def step(j, c2):
            pltpu.async_copy(z.at[sbuf.at[j]], rows, sem).wait()
            pltpu.sync_copy(rows, acc.at[dbuf.at[j]], add=True)
            return c2

        lax.fori_loop(0, IB, step, 0)
        return carry

    lax.fori_loop(0, NSUP, super_step, 0)
    plsc.subcore_barrier()

    pltpu.sync_copy(acc.at[pl.ds(base, ROWS_PER_TILE)],
                    p_out.at[cid, pl.ds(base, ROWS_PER_TILE)])


def _sc_cnt_body(dstr, zbig, ones_w, c_out, acc, dbuf, ones_v):
    cid = lax.axis_index("c")
    sid = lax.axis_index("s")
    wid = cid * NS + sid
    base = sid * ROWS_PER_TILE

    pltpu.sync_copy(zbig.at[pl.ds(base, ROWS_PER_TILE)],
                    acc.at[pl.ds(base, ROWS_PER_TILE)])
    pltpu.sync_copy(ones_w, ones_v)
    plsc.subcore_barrier()

    def super_step(g, carry):
        pltpu.sync_copy(dstr.at[wid, pl.ds(g * IB, IB)], dbuf)

        def step(j, c2):
            pltpu.sync_copy(ones_v, acc.at[dbuf.at[j]], add=True)
            return c2

        lax.fori_loop(0, IB, step, 0)
        return carry

    lax.fori_loop(0, NSUP, super_step, 0)
    plsc.subcore_barrier()
    pltpu.sync_copy(acc.at[pl.ds(base, ROWS_PER_TILE)],
                    c_out.at[cid, pl.ds(base, ROWS_PER_TILE)])


_SC_MESH = plsc.VectorSubcoreMesh(core_axis_name="c", subcore_axis_name="s")

_sc_agg = pl.kernel(
    _sc_agg_body,
    out_type=[jax.ShapeDtypeStruct((NC, N_PAD, D), jnp.float32)],
    mesh=_SC_MESH,
    scratch_types=[
        pltpu.VMEM_SHARED((N_PAD, D), jnp.float32),
        pltpu.VMEM((IB, CH), jnp.int32),
        pltpu.VMEM((IB, CH), jnp.int32),
        pltpu.VMEM((CH, D), jnp.float32),
        pltpu.SemaphoreType.DMA,
    ],
)

_sc_cnt = pl.kernel(
    _sc_cnt_body,
    out_type=[jax.ShapeDtypeStruct((NC, N_PAD, D), jnp.float32)],
    mesh=_SC_MESH,
    scratch_types=[
        pltpu.VMEM_SHARED((N_PAD, D), jnp.float32),
        pltpu.VMEM((IB, CH), jnp.int32),
        pltpu.VMEM((CH, D), jnp.float32),
    ],
)


def kernel(x, edge_index, W_proj, b_proj, Wl1, bl1, Wr1, Wl2, bl2, Wr2,
           Wl3, bl3, Wr3, W_cls, b_cls):
    f32 = jnp.float32
    src = edge_index[0]
    dst = edge_index[1]
    pad = E_PAD - E
    srcr = jnp.concatenate([src, jnp.zeros((pad,), jnp.int32)]
                           ).reshape(NW, STEPS, CH)
    dstr = jnp.concatenate([dst, jnp.full((pad,), N, jnp.int32)]
                           ).reshape(NW, STEPS, CH)

    zbig = jnp.zeros((N_PAD, D), f32)
    ones_w = jnp.ones((CH, D), f32)

    wpT = W_proj.T
    bp = b_proj.reshape(1, D)
    wl1T, wr1T = Wl1.T, Wr1.T
    wl2T, wr2T = Wl2.T, Wr2.T
    wl3T, wr3T = Wl3.T, Wr3.T
    bl1r, bl2r, bl3r = bl1.reshape(1, D), bl2.reshape(1, D), bl3.reshape(1, D)
    wcT = jnp.zeros((D, D), f32).at[:, :NCLS].set(W_cls.T)
    bc = jnp.zeros((1, D), f32).at[0, :NCLS].set(b_cls)

    (cnts,) = _sc_cnt(dstr, zbig, ones_w)
    z1, r1 = _tc_pre(x, wpT, bp, wl1T, wr1T)
    (p1,) = _sc_agg(z1, srcr, dstr, zbig)
    z2, r2 = _tc_mid(p1, cnts, r1, bl1r, wl2T, wr2T)
    (p2,) = _sc_agg(z2, srcr, dstr, zbig)
    z3, r3 = _tc_mid(p2, cnts, r2, bl2r, wl3T, wr3T)
    (p3,) = _sc_agg(z3, srcr, dstr, zbig)
    out_pad = _tc_fin(p3, cnts, r3, bl3r, wcT, bc)
    return out_pad[:, :NCLS]


# speedup vs baseline: 6.8128x; 2.2440x over previous
import functools
import jax
import jax.numpy as jnp
from jax import lax
from jax.experimental import pallas as pl
from jax.experimental.pallas import tpu as pltpu
from jax.experimental.pallas import tpu_sc as plsc

N = 10000
E = 320000
D = 128
NCLS = 40

NC = 2
NS = 16
NW = NC * NS
CH = 128
STEPS = 80
IB = 16
NSUP = STEPS // IB
E_PAD = NW * STEPS * CH
N_PAD = 10112
ROWS_PER_TILE = N_PAD // NS
TCB = 1000


def _pre_body(x_ref, wpT, bp, wlT, wrT, z_ref, r_ref):
    h = jnp.maximum(jnp.dot(x_ref[...], wpT[...],
                            preferred_element_type=jnp.float32) + bp[...], 0.0)
    z_ref[...] = jnp.dot(h, wlT[...], preferred_element_type=jnp.float32)
    r_ref[...] = jnp.dot(h, wrT[...], preferred_element_type=jnp.float32)


def _mid_body(p_ref, c_ref, r_ref, bl, wlT, wrT, z_ref, rn_ref):
    p = p_ref[0] + p_ref[1]
    c = c_ref[0, :, 0:1] + c_ref[1, :, 0:1]
    inv = 1.0 / jnp.maximum(c, 1.0)
    h = jnp.maximum(p * inv + bl[...] + r_ref[...], 0.0)
    z_ref[...] = jnp.dot(h, wlT[...], preferred_element_type=jnp.float32)
    rn_ref[...] = jnp.dot(h, wrT[...], preferred_element_type=jnp.float32)


def _fin_body(p_ref, c_ref, r_ref, bl, wcT, bc, o_ref):
    p = p_ref[0] + p_ref[1]
    c = c_ref[0, :, 0:1] + c_ref[1, :, 0:1]
    inv = 1.0 / jnp.maximum(c, 1.0)
    h = jnp.maximum(p * inv + bl[...] + r_ref[...], 0.0)
    o_ref[...] = jnp.dot(h, wcT[...], preferred_element_type=jnp.float32) + bc[...]


def _row_spec():
    return pl.BlockSpec((TCB, D), lambda i: (i, 0))


def _full_spec(shape):
    return pl.BlockSpec(shape, lambda i: tuple(0 for _ in shape))


def _part_spec():
    return pl.BlockSpec((2, TCB, D), lambda i: (0, i, 0))


def _cnt_spec():
    return pl.BlockSpec((2, TCB, D), lambda i: (0, i, 0))


_GRID = N // TCB


def _tc_pre(x, wpT, bp, wlT, wrT):
    return pl.pallas_call(
        _pre_body,
        grid=(_GRID,),
        in_specs=[_row_spec(), _full_spec((D, D)), _full_spec((1, D)),
                  _full_spec((D, D)), _full_spec((D, D))],
        out_specs=[_row_spec(), _row_spec()],
        out_shape=[jax.ShapeDtypeStruct((N, D), jnp.float32),
                   jax.ShapeDtypeStruct((N, D), jnp.float32)],
    )(x, wpT, bp, wlT, wrT)


def _tc_mid(parts, cnts, r, bl, wlT, wrT):
    return pl.pallas_call(
        _mid_body,
        grid=(_GRID,),
        in_specs=[_part_spec(), _cnt_spec(), _row_spec(), _full_spec((1, D)),
                  _full_spec((D, D)), _full_spec((D, D))],
        out_specs=[_row_spec(), _row_spec()],
        out_shape=[jax.ShapeDtypeStruct((N, D), jnp.float32),
                   jax.ShapeDtypeStruct((N, D), jnp.float32)],
    )(parts, cnts, r, bl, wlT, wrT)


def _tc_fin(parts, cnts, r, bl, wcT, bc):
    return pl.pallas_call(
        _fin_body,
        grid=(_GRID,),
        in_specs=[_part_spec(), _cnt_spec(), _row_spec(), _full_spec((1, D)),
                  _full_spec((D, D)), _full_spec((1, D))],
        out_specs=_row_spec(),
        out_shape=jax.ShapeDtypeStruct((N, D), jnp.float32),
    )(parts, cnts, r, bl, wcT, bc)


def _sc_agg_body(z, srcr, dstr, zbig, p_out, acc, sbuf, dbuf, rows, sem):
    cid = lax.axis_index("c")
    sid = lax.axis_index("s")
    wid = cid * NS + sid
    base = sid * ROWS_PER_TILE

    pltpu.sync_copy(zbig.at[pl.ds(base, ROWS_PER_TILE)],
                    acc.at[pl.ds(base, ROWS_PER_TILE)])
    plsc.subcore_barrier()

    def super_step(g, carry):
        pltpu.sync_copy(srcr.at[wid, pl.ds(g * IB, IB)], sbuf)
        pltpu.sync_copy(dstr.at[wid, pl.ds(g * IB, IB)], dbuf)

        def step(j, c2):
            pltpu.async_copy(z.at[sbuf.at[j]], rows, sem).wait()
            pltpu.sync_copy(rows, acc.at[dbuf.at[j]], add=True)
            return c2

        lax.fori_loop(0, IB, step, 0)
        return carry

    lax.fori_loop(0, NSUP, super_step, 0)
    plsc.subcore_barrier()

    pltpu.sync_copy(acc.at[pl.ds(base, ROWS_PER_TILE)],
                    p_out.at[cid, pl.ds(base, ROWS_PER_TILE)])


def _sc_cnt_body(dstr, zbig, ones_w, c_out, acc, dbuf, ones_v):
    cid = lax.axis_index("c")
    sid = lax.axis_index("s")
    wid = cid * NS + sid
    base = sid * ROWS_PER_TILE

    pltpu.sync_copy(zbig.at[pl.ds(base, ROWS_PER_TILE)],
                    acc.at[pl.ds(base, ROWS_PER_TILE)])
    pltpu.sync_copy(ones_w, ones_v)
    plsc.subcore_barrier()

    def super_step(g, carry):
        pltpu.sync_copy(dstr.at[wid, pl.ds(g * IB, IB)], dbuf)

        def step(j, c2):
            pltpu.sync_copy(ones_v, acc.at[dbuf.at[j]], add=True)
            return c2

        lax.fori_loop(0, IB, step, 0)
        return carry

    lax.fori_loop(0, NSUP, super_step, 0)
    plsc.subcore_barrier()
    pltpu.sync_copy(acc.at[pl.ds(base, ROWS_PER_TILE)],
                    c_out.at[cid, pl.ds(base, ROWS_PER_TILE)])


_SC_MESH = plsc.VectorSubcoreMesh(core_axis_name="c", subcore_axis_name="s")

_sc_agg = pl.kernel(
    _sc_agg_body,
    out_type=[jax.ShapeDtypeStruct((NC, N_PAD, D), jnp.float32)],
    mesh=_SC_MESH,
    scratch_types=[
        pltpu.VMEM_SHARED((N_PAD, D), jnp.float32),
        pltpu.VMEM((IB, CH), jnp.int32),
        pltpu.VMEM((IB, CH), jnp.int32),
        pltpu.VMEM((CH, D), jnp.float32),
        pltpu.SemaphoreType.DMA,
    ],
)

_sc_cnt = pl.kernel(
    _sc_cnt_body,
    out_type=[jax.ShapeDtypeStruct((NC, N_PAD, D), jnp.float32)],
    mesh=_SC_MESH,
    scratch_types=[
        pltpu.VMEM_SHARED((N_PAD, D), jnp.float32),
        pltpu.VMEM((IB, CH), jnp.int32),
        pltpu.VMEM((CH, D), jnp.float32),
    ],
)


def kernel(x, edge_index, W_proj, b_proj, Wl1, bl1, Wr1, Wl2, bl2, Wr2,
           Wl3, bl3, Wr3, W_cls, b_cls):
    f32 = jnp.float32
    src = edge_index[0]
    dst = edge_index[1]
    pad = E_PAD - E
    pad_src = jnp.arange(pad, dtype=jnp.int32) % N
    srcr = jnp.concatenate([src, pad_src]).reshape(NW, STEPS, CH)
    dstr = jnp.concatenate([dst, jnp.full((pad,), N, jnp.int32)]
                           ).reshape(NW, STEPS, CH)

    zbig = jnp.zeros((N_PAD, D), f32)
    ones_w = jnp.ones((CH, D), f32)

    wpT = W_proj.T
    bp = b_proj.reshape(1, D)
    wl1T, wr1T = Wl1.T, Wr1.T
    wl2T, wr2T = Wl2.T, Wr2.T
    wl3T, wr3T = Wl3.T, Wr3.T
    bl1r, bl2r, bl3r = bl1.reshape(1, D), bl2.reshape(1, D), bl3.reshape(1, D)
    wcT = jnp.zeros((D, D), f32).at[:, :NCLS].set(W_cls.T)
    bc = jnp.zeros((1, D), f32).at[0, :NCLS].set(b_cls)

    (cnts,) = _sc_cnt(dstr, zbig, ones_w)
    z1, r1 = _tc_pre(x, wpT, bp, wl1T, wr1T)
    (p1,) = _sc_agg(z1, srcr, dstr, zbig)
    z2, r2 = _tc_mid(p1, cnts, r1, bl1r, wl2T, wr2T)
    (p2,) = _sc_agg(z2, srcr, dstr, zbig)
    z3, r3 = _tc_mid(p2, cnts, r2, bl2r, wl3T, wr3T)
    (p3,) = _sc_agg(z3, srcr, dstr, zbig)
    out_pad = _tc_fin(p3, cnts, r3, bl3r, wcT, bc)
    return out_pad[:, :NCLS]


# speedup vs baseline: 8.3203x; 1.2213x over previous
import functools
import jax
import jax.numpy as jnp
from jax import lax
from jax.experimental import pallas as pl
from jax.experimental.pallas import tpu as pltpu
from jax.experimental.pallas import tpu_sc as plsc

N = 10000
E = 320000
D = 128
NCLS = 40

NC = 2
NS = 16
NW = NC * NS
CH = 128
STEPS = 80
IB = 16
NSUP = STEPS // IB
E_PAD = NW * STEPS * CH
N_PAD = 10112
ROWS_PER_TILE = N_PAD // NS
TCB = 1000


def _pre_body(x_ref, wpT, bp, wlT, wrT, z_ref, r_ref):
    h = jnp.maximum(jnp.dot(x_ref[...], wpT[...],
                            preferred_element_type=jnp.float32) + bp[...], 0.0)
    z_ref[...] = jnp.dot(h, wlT[...], preferred_element_type=jnp.float32)
    r_ref[...] = jnp.dot(h, wrT[...], preferred_element_type=jnp.float32)


def _mid_body(p_ref, c_ref, r_ref, bl, wlT, wrT, z_ref, rn_ref):
    p = p_ref[0] + p_ref[1]
    c = c_ref[0, :, 0:1] + c_ref[1, :, 0:1]
    inv = 1.0 / jnp.maximum(c, 1.0)
    h = jnp.maximum(p * inv + bl[...] + r_ref[...], 0.0)
    z_ref[...] = jnp.dot(h, wlT[...], preferred_element_type=jnp.float32)
    rn_ref[...] = jnp.dot(h, wrT[...], preferred_element_type=jnp.float32)


def _fin_body(p_ref, c_ref, r_ref, bl, wcT, bc, o_ref):
    p = p_ref[0] + p_ref[1]
    c = c_ref[0, :, 0:1] + c_ref[1, :, 0:1]
    inv = 1.0 / jnp.maximum(c, 1.0)
    h = jnp.maximum(p * inv + bl[...] + r_ref[...], 0.0)
    o_ref[...] = jnp.dot(h, wcT[...], preferred_element_type=jnp.float32) + bc[...]


def _row_spec():
    return pl.BlockSpec((TCB, D), lambda i: (i, 0))


def _full_spec(shape):
    return pl.BlockSpec(shape, lambda i: tuple(0 for _ in shape))


def _part_spec():
    return pl.BlockSpec((2, TCB, D), lambda i: (0, i, 0))


def _cnt_spec():
    return pl.BlockSpec((2, TCB, D), lambda i: (0, i, 0))


_GRID = N // TCB


def _tc_pre(x, wpT, bp, wlT, wrT):
    return pl.pallas_call(
        _pre_body,
        grid=(_GRID,),
        in_specs=[_row_spec(), _full_spec((D, D)), _full_spec((1, D)),
                  _full_spec((D, D)), _full_spec((D, D))],
        out_specs=[_row_spec(), _row_spec()],
        out_shape=[jax.ShapeDtypeStruct((N, D), jnp.float32),
                   jax.ShapeDtypeStruct((N, D), jnp.float32)],
    )(x, wpT, bp, wlT, wrT)


def _tc_mid(parts, cnts, r, bl, wlT, wrT):
    return pl.pallas_call(
        _mid_body,
        grid=(_GRID,),
        in_specs=[_part_spec(), _cnt_spec(), _row_spec(), _full_spec((1, D)),
                  _full_spec((D, D)), _full_spec((D, D))],
        out_specs=[_row_spec(), _row_spec()],
        out_shape=[jax.ShapeDtypeStruct((N, D), jnp.float32),
                   jax.ShapeDtypeStruct((N, D), jnp.float32)],
    )(parts, cnts, r, bl, wlT, wrT)


def _tc_fin(parts, cnts, r, bl, wcT, bc):
    return pl.pallas_call(
        _fin_body,
        grid=(_GRID,),
        in_specs=[_part_spec(), _cnt_spec(), _row_spec(), _full_spec((1, D)),
                  _full_spec((D, D)), _full_spec((1, D))],
        out_specs=_row_spec(),
        out_shape=jax.ShapeDtypeStruct((N, D), jnp.float32),
    )(parts, cnts, r, bl, wcT, bc)


def _sc_agg_body(z, srcr, dstr, zbig, p_out, acc, sbuf, dbuf, rows, sem):
    cid = lax.axis_index("c")
    sid = lax.axis_index("s")
    wid = cid * NS + sid
    base = sid * ROWS_PER_TILE

    pltpu.sync_copy(zbig.at[pl.ds(base, ROWS_PER_TILE)],
                    acc.at[pl.ds(base, ROWS_PER_TILE)])
    plsc.subcore_barrier()

    def super_step(g, carry):
        pltpu.sync_copy(srcr.at[wid, pl.ds(g * IB, IB)], sbuf)
        pltpu.sync_copy(dstr.at[wid, pl.ds(g * IB, IB)], dbuf)

        rA = rows.at[0]
        rB = rows.at[1]
        pltpu.async_copy(z.at[sbuf.at[0]], rA, sem)

        def pair(i, c2):
            j0 = 2 * i
            j1 = j0 + 1
            pltpu.make_async_copy(z.at[sbuf.at[j0]], rA, sem).wait()
            pltpu.async_copy(z.at[sbuf.at[j1]], rB, sem)
            pltpu.sync_copy(rA, acc.at[dbuf.at[j0]], add=True)
            pltpu.make_async_copy(z.at[sbuf.at[j1]], rB, sem).wait()

            @pl.when(i < IB // 2 - 1)
            def _():
                pltpu.async_copy(z.at[sbuf.at[j0 + 2]], rA, sem)

            pltpu.sync_copy(rB, acc.at[dbuf.at[j1]], add=True)
            return c2

        lax.fori_loop(0, IB // 2, pair, 0)
        return carry

    lax.fori_loop(0, NSUP, super_step, 0)
    plsc.subcore_barrier()

    pltpu.sync_copy(acc.at[pl.ds(base, ROWS_PER_TILE)],
                    p_out.at[cid, pl.ds(base, ROWS_PER_TILE)])


def _sc_cnt_body(dstr, zbig, ones_w, c_out, acc, dbuf, ones_v):
    cid = lax.axis_index("c")
    sid = lax.axis_index("s")
    wid = cid * NS + sid
    base = sid * ROWS_PER_TILE

    pltpu.sync_copy(zbig.at[pl.ds(base, ROWS_PER_TILE)],
                    acc.at[pl.ds(base, ROWS_PER_TILE)])
    pltpu.sync_copy(ones_w, ones_v)
    plsc.subcore_barrier()

    def super_step(g, carry):
        pltpu.sync_copy(dstr.at[wid, pl.ds(g * IB, IB)], dbuf)

        def step(j, c2):
            pltpu.sync_copy(ones_v, acc.at[dbuf.at[j]], add=True)
            return c2

        lax.fori_loop(0, IB, step, 0)
        return carry

    lax.fori_loop(0, NSUP, super_step, 0)
    plsc.subcore_barrier()
    pltpu.sync_copy(acc.at[pl.ds(base, ROWS_PER_TILE)],
                    c_out.at[cid, pl.ds(base, ROWS_PER_TILE)])


_SC_MESH = plsc.VectorSubcoreMesh(core_axis_name="c", subcore_axis_name="s")

_sc_agg = pl.kernel(
    _sc_agg_body,
    out_type=[jax.ShapeDtypeStruct((NC, N_PAD, D), jnp.float32)],
    mesh=_SC_MESH,
    scratch_types=[
        pltpu.VMEM_SHARED((N_PAD, D), jnp.float32),
        pltpu.VMEM((IB, CH), jnp.int32),
        pltpu.VMEM((IB, CH), jnp.int32),
        pltpu.VMEM((2, CH, D), jnp.float32),
        pltpu.SemaphoreType.DMA,
    ],
)

_sc_cnt = pl.kernel(
    _sc_cnt_body,
    out_type=[jax.ShapeDtypeStruct((NC, N_PAD, D), jnp.float32)],
    mesh=_SC_MESH,
    scratch_types=[
        pltpu.VMEM_SHARED((N_PAD, D), jnp.float32),
        pltpu.VMEM((IB, CH), jnp.int32),
        pltpu.VMEM((CH, D), jnp.float32),
    ],
)


def kernel(x, edge_index, W_proj, b_proj, Wl1, bl1, Wr1, Wl2, bl2, Wr2,
           Wl3, bl3, Wr3, W_cls, b_cls):
    f32 = jnp.float32
    src = edge_index[0]
    dst = edge_index[1]
    pad = E_PAD - E
    pad_src = jnp.arange(pad, dtype=jnp.int32) % N
    srcr = jnp.concatenate([src, pad_src]).reshape(NW, STEPS, CH)
    dstr = jnp.concatenate([dst, jnp.full((pad,), N, jnp.int32)]
                           ).reshape(NW, STEPS, CH)

    zbig = jnp.zeros((N_PAD, D), f32)
    ones_w = jnp.ones((CH, D), f32)

    wpT = W_proj.T
    bp = b_proj.reshape(1, D)
    wl1T, wr1T = Wl1.T, Wr1.T
    wl2T, wr2T = Wl2.T, Wr2.T
    wl3T, wr3T = Wl3.T, Wr3.T
    bl1r, bl2r, bl3r = bl1.reshape(1, D), bl2.reshape(1, D), bl3.reshape(1, D)
    wcT = jnp.zeros((D, D), f32).at[:, :NCLS].set(W_cls.T)
    bc = jnp.zeros((1, D), f32).at[0, :NCLS].set(b_cls)

    (cnts,) = _sc_cnt(dstr, zbig, ones_w)
    z1, r1 = _tc_pre(x, wpT, bp, wl1T, wr1T)
    (p1,) = _sc_agg(z1, srcr, dstr, zbig)
    z2, r2 = _tc_mid(p1, cnts, r1, bl1r, wl2T, wr2T)
    (p2,) = _sc_agg(z2, srcr, dstr, zbig)
    z3, r3 = _tc_mid(p2, cnts, r2, bl2r, wl3T, wr3T)
    (p3,) = _sc_agg(z3, srcr, dstr, zbig)
    out_pad = _tc_fin(p3, cnts, r3, bl3r, wcT, bc)
    return out_pad[:, :NCLS]


# speedup vs baseline: 9.2257x; 1.1088x over previous
import functools
import jax
import jax.numpy as jnp
from jax import lax
from jax.experimental import pallas as pl
from jax.experimental.pallas import tpu as pltpu
from jax.experimental.pallas import tpu_sc as plsc

N = 10000
E = 320000
D = 128
NCLS = 40

NC = 2
NS = 16
NW = NC * NS
CH = 128
STEPS = 80
IB = 16
NSUP = STEPS // IB
CH2 = 64
STEPS2 = 160
IB2 = 32
NSUP2 = STEPS2 // IB2
E_PAD = NW * STEPS * CH
N_PAD = 10112
ROWS_PER_TILE = N_PAD // NS
TCB = 1000


def _pre_body(x_ref, wpT, bp, wlT, wrT, z_ref, r_ref):
    h = jnp.maximum(jnp.dot(x_ref[...], wpT[...],
                            preferred_element_type=jnp.float32) + bp[...], 0.0)
    z_ref[...] = jnp.dot(h, wlT[...], preferred_element_type=jnp.float32)
    r_ref[...] = jnp.dot(h, wrT[...], preferred_element_type=jnp.float32)


def _mid_body(p_ref, c_ref, r_ref, bl, wlT, wrT, z_ref, rn_ref):
    p = p_ref[0] + p_ref[1]
    c = c_ref[0, :, 0:1] + c_ref[1, :, 0:1]
    inv = 1.0 / jnp.maximum(c, 1.0)
    h = jnp.maximum(p * inv + bl[...] + r_ref[...], 0.0)
    z_ref[...] = jnp.dot(h, wlT[...], preferred_element_type=jnp.float32)
    rn_ref[...] = jnp.dot(h, wrT[...], preferred_element_type=jnp.float32)


def _fin_body(p_ref, c_ref, r_ref, bl, wcT, bc, o_ref):
    p = p_ref[0] + p_ref[1]
    c = c_ref[0, :, 0:1] + c_ref[1, :, 0:1]
    inv = 1.0 / jnp.maximum(c, 1.0)
    h = jnp.maximum(p * inv + bl[...] + r_ref[...], 0.0)
    o_ref[...] = jnp.dot(h, wcT[...], preferred_element_type=jnp.float32) + bc[...]


def _row_spec():
    return pl.BlockSpec((TCB, D), lambda i: (i, 0))


def _full_spec(shape):
    return pl.BlockSpec(shape, lambda i: tuple(0 for _ in shape))


def _part_spec():
    return pl.BlockSpec((2, TCB, D), lambda i: (0, i, 0))


def _cnt_spec():
    return pl.BlockSpec((2, TCB, D), lambda i: (0, i, 0))


_GRID = N // TCB


def _tc_pre(x, wpT, bp, wlT, wrT):
    return pl.pallas_call(
        _pre_body,
        grid=(_GRID,),
        in_specs=[_row_spec(), _full_spec((D, D)), _full_spec((1, D)),
                  _full_spec((D, D)), _full_spec((D, D))],
        out_specs=[_row_spec(), _row_spec()],
        out_shape=[jax.ShapeDtypeStruct((N, D), jnp.float32),
                   jax.ShapeDtypeStruct((N, D), jnp.float32)],
    )(x, wpT, bp, wlT, wrT)


def _tc_mid(parts, cnts, r, bl, wlT, wrT):
    return pl.pallas_call(
        _mid_body,
        grid=(_GRID,),
        in_specs=[_part_spec(), _cnt_spec(), _row_spec(), _full_spec((1, D)),
                  _full_spec((D, D)), _full_spec((D, D))],
        out_specs=[_row_spec(), _row_spec()],
        out_shape=[jax.ShapeDtypeStruct((N, D), jnp.float32),
                   jax.ShapeDtypeStruct((N, D), jnp.float32)],
    )(parts, cnts, r, bl, wlT, wrT)


def _tc_fin(parts, cnts, r, bl, wcT, bc):
    return pl.pallas_call(
        _fin_body,
        grid=(_GRID,),
        in_specs=[_part_spec(), _cnt_spec(), _row_spec(), _full_spec((1, D)),
                  _full_spec((D, D)), _full_spec((1, D))],
        out_specs=_row_spec(),
        out_shape=jax.ShapeDtypeStruct((N, D), jnp.float32),
    )(parts, cnts, r, bl, wcT, bc)


def _sc_agg_body(z, srcr, dstr, zbig, p_out, acc, sbuf, dbuf, rows,
                 sem_g, sem_s):
    cid = lax.axis_index("c")
    sid = lax.axis_index("s")
    wid = cid * NS + sid
    base = sid * ROWS_PER_TILE

    pltpu.sync_copy(zbig.at[pl.ds(base, ROWS_PER_TILE)],
                    acc.at[pl.ds(base, ROWS_PER_TILE)])
    plsc.subcore_barrier()

    bufs = [rows.at[b] for b in range(4)]

    def super_step(g, carry):
        pltpu.sync_copy(srcr.at[wid, pl.ds(g * IB2, IB2)], sbuf)
        pltpu.sync_copy(dstr.at[wid, pl.ds(g * IB2, IB2)], dbuf)

        pltpu.async_copy(z.at[sbuf.at[0]], bufs[0], sem_g)
        pltpu.async_copy(z.at[sbuf.at[1]], bufs[1], sem_g)

        def quad(i, c2):
            j0 = 4 * i
            for k in range(4):
                j = j0 + k
                b = bufs[k]
                bn = bufs[(k + 2) % 4]
                pltpu.make_async_copy(z.at[sbuf.at[j]], b, sem_g).wait()
                pltpu.async_copy(b, acc.at[dbuf.at[j]], sem_s, add=True)

                @pl.when(j >= 2)
                def _():
                    pltpu.make_async_copy(bn, acc.at[dbuf.at[j]],
                                          sem_s).wait()

                @pl.when(j + 2 < IB2)
                def _():
                    pltpu.async_copy(z.at[sbuf.at[j + 2]], bn, sem_g)
            return c2

        lax.fori_loop(0, IB2 // 4, quad, 0)
        pltpu.make_async_copy(bufs[2], acc.at[dbuf.at[IB2 - 2]],
                              sem_s).wait()
        pltpu.make_async_copy(bufs[3], acc.at[dbuf.at[IB2 - 1]],
                              sem_s).wait()
        return carry

    lax.fori_loop(0, NSUP2, super_step, 0)
    plsc.subcore_barrier()

    pltpu.sync_copy(acc.at[pl.ds(base, ROWS_PER_TILE)],
                    p_out.at[cid, pl.ds(base, ROWS_PER_TILE)])


def _sc_cnt_body(dstr, zbig, ones_w, c_out, acc, dbuf, ones_v):
    cid = lax.axis_index("c")
    sid = lax.axis_index("s")
    wid = cid * NS + sid
    base = sid * ROWS_PER_TILE

    pltpu.sync_copy(zbig.at[pl.ds(base, ROWS_PER_TILE)],
                    acc.at[pl.ds(base, ROWS_PER_TILE)])
    pltpu.sync_copy(ones_w, ones_v)
    plsc.subcore_barrier()

    def super_step(g, carry):
        pltpu.sync_copy(dstr.at[wid, pl.ds(g * IB, IB)], dbuf)

        def step(j, c2):
            pltpu.sync_copy(ones_v, acc.at[dbuf.at[j]], add=True)
            return c2

        lax.fori_loop(0, IB, step, 0)
        return carry

    lax.fori_loop(0, NSUP, super_step, 0)
    plsc.subcore_barrier()
    pltpu.sync_copy(acc.at[pl.ds(base, ROWS_PER_TILE)],
                    c_out.at[cid, pl.ds(base, ROWS_PER_TILE)])


_SC_MESH = plsc.VectorSubcoreMesh(core_axis_name="c", subcore_axis_name="s")

_sc_agg = pl.kernel(
    _sc_agg_body,
    out_type=[jax.ShapeDtypeStruct((NC, N_PAD, D), jnp.float32)],
    mesh=_SC_MESH,
    scratch_types=[
        pltpu.VMEM_SHARED((N_PAD, D), jnp.float32),
        pltpu.VMEM((IB2, CH2), jnp.int32),
        pltpu.VMEM((IB2, CH2), jnp.int32),
        pltpu.VMEM((4, CH2, D), jnp.float32),
        pltpu.SemaphoreType.DMA,
        pltpu.SemaphoreType.DMA,
    ],
)

_sc_cnt = pl.kernel(
    _sc_cnt_body,
    out_type=[jax.ShapeDtypeStruct((NC, N_PAD, D), jnp.float32)],
    mesh=_SC_MESH,
    scratch_types=[
        pltpu.VMEM_SHARED((N_PAD, D), jnp.float32),
        pltpu.VMEM((IB, CH), jnp.int32),
        pltpu.VMEM((CH, D), jnp.float32),
    ],
)


def kernel(x, edge_index, W_proj, b_proj, Wl1, bl1, Wr1, Wl2, bl2, Wr2,
           Wl3, bl3, Wr3, W_cls, b_cls):
    f32 = jnp.float32
    src = edge_index[0]
    dst = edge_index[1]
    pad = E_PAD - E
    pad_src = jnp.arange(pad, dtype=jnp.int32) % N
    src_p = jnp.concatenate([src, pad_src])
    dst_p = jnp.concatenate([dst, jnp.full((pad,), N, jnp.int32)])
    srcr = src_p.reshape(NW, STEPS2, CH2)
    dstr = dst_p.reshape(NW, STEPS2, CH2)
    dstr_c = dst_p.reshape(NW, STEPS, CH)

    zbig = jnp.zeros((N_PAD, D), f32)
    ones_w = jnp.ones((CH, D), f32)

    wpT = W_proj.T
    bp = b_proj.reshape(1, D)
    wl1T, wr1T = Wl1.T, Wr1.T
    wl2T, wr2T = Wl2.T, Wr2.T
    wl3T, wr3T = Wl3.T, Wr3.T
    bl1r, bl2r, bl3r = bl1.reshape(1, D), bl2.reshape(1, D), bl3.reshape(1, D)
    wcT = jnp.zeros((D, D), f32).at[:, :NCLS].set(W_cls.T)
    bc = jnp.zeros((1, D), f32).at[0, :NCLS].set(b_cls)

    (cnts,) = _sc_cnt(dstr_c, zbig, ones_w)
    z1, r1 = _tc_pre(x, wpT, bp, wl1T, wr1T)
    (p1,) = _sc_agg(z1, srcr, dstr, zbig)
    z2, r2 = _tc_mid(p1, cnts, r1, bl1r, wl2T, wr2T)
    (p2,) = _sc_agg(z2, srcr, dstr, zbig)
    z3, r3 = _tc_mid(p2, cnts, r2, bl2r, wl3T, wr3T)
    (p3,) = _sc_agg(z3, srcr, dstr, zbig)
    out_pad = _tc_fin(p3, cnts, r3, bl3r, wcT, bc)
    return out_pad[:, :NCLS]
